# baseline (speedup 1.0000x reference)
import functools

import jax
import jax.numpy as jnp
from jax import lax
from jax.experimental import pallas as pl
from jax.experimental.pallas import tpu as pltpu
from jax.experimental.pallas import tpu_sc as plsc

_B = 128
_N = 32768
_NW = 32
_RPW = _B // _NW
_L = 16
_NCH = _N // _L

_LEVELS = ((24, 7), (18, 6), (12, 6), (6, 6), (0, 6))
_MAXBINS = 1 << 7


def _lane_iota():
    return lax.iota(jnp.int32, _L)


def _scalar(v):
    return jnp.sum(jnp.where(_lane_iota() == 0, v, jnp.int32(0)))


def _body(scores_hbm, k_hbm, out_hbm, row_v, colla_v, collb_v, hist_v,
          tot_v, kv_v):
    nc = 2
    wid = lax.axis_index("s") * nc + lax.axis_index("c")
    pltpu.sync_copy(k_hbm, kv_v)
    k0 = _scalar(kv_v[...])
    iota = _lane_iota()
    ones = jnp.ones((_L,), jnp.int32)

    def histogram_scan(nbins, k_cur):
        nch = nbins // _L

        def merge_chunk(c, _):
            acc = jnp.zeros((_L,), jnp.int32)

            def add_lane(l, acc):
                return acc + hist_v[pl.ds(l * nbins + c * _L, _L)]

            acc = lax.fori_loop(0, _L, add_lane, acc)
            tot_v[pl.ds(c * _L, _L)] = acc
            return 0

        lax.fori_loop(0, nch, merge_chunk, 0)

        def scan_chunk(j, carry):
            after, best_d, best_k = carry
            c = nch - 1 - j
            v = tot_v[pl.ds(c * _L, _L)]
            cs = plsc.cumsum(v)
            tot = jnp.sum(v)
            suffix_incl = after + tot - cs + v
            feas = suffix_incl >= k_cur
            lane = jnp.sum(feas.astype(jnp.int32)) - 1
            above = jnp.sum(jnp.where(iota == lane, suffix_incl - v,
                                      jnp.int32(0)))
            found = (lane >= 0) & (best_d < 0)
            best_d = jnp.where(found, c * _L + lane, best_d)
            best_k = jnp.where(found, k_cur - above, best_k)
            return after + tot, best_d, best_k

        _, d, k_new = lax.fori_loop(
            0, nch, scan_chunk,
            (jnp.int32(0), jnp.int32(-1), jnp.int32(0)))
        return d, k_new

    def zero_hist(nbins):
        def z(i, _):
            hist_v[pl.ds(i * _L, _L)] = jnp.zeros((_L,), jnp.int32)
            return 0

        lax.fori_loop(0, (nbins * _L) // _L, z, 0)

    def do_row(r, _):
        row = wid * _RPW + r
        pltpu.sync_copy(scores_hbm.at[row], row_v)

        sh0, w0 = _LEVELS[0]
        nbins0 = 1 << w0
        zero_hist(nbins0)

        def l1_hist(i, _):
            x = row_v[pl.ds(i * _L, _L)]
            bits = plsc.bitcast(x, jnp.int32) & jnp.int32(0x7FFFFFFF)
            digit = lax.shift_right_logical(bits, jnp.int32(sh0))
            plsc.addupdate_scatter(hist_v, [iota * nbins0 + digit], ones)
            return 0

        lax.fori_loop(0, _NCH, l1_hist, 0)
        d1, k_cur = histogram_scan(nbins0, k0)
        prefix = d1

        def l1_collect(i, off):
            x = row_v[pl.ds(i * _L, _L)]
            bits = plsc.bitcast(x, jnp.int32) & jnp.int32(0x7FFFFFFF)
            match = lax.shift_right_logical(bits, jnp.int32(sh0)) == prefix
            mi = match.astype(jnp.int32)
            cs = plsc.cumsum(mi)
            pos = off + cs - mi
            plsc.store_scatter(colla_v, [pos], bits, mask=match)
            return off + jnp.sum(mi)

        m = lax.fori_loop(0, _NCH, l1_collect, jnp.int32(0))

        src, dst = colla_v, collb_v
        for lvl in range(1, len(_LEVELS)):
            sh, w = _LEVELS[lvl]
            nbins = 1 << w
            zero_hist(nbins)
            nit = (m + (_L - 1)) // _L

            def lv_hist(i, _, src=src, sh=sh, nbins=nbins, m=m):
                b = src[pl.ds(i * _L, _L)]
                valid = (i * _L + iota) < m
                digit = lax.shift_right_logical(b, jnp.int32(sh)) \
                    & jnp.int32(nbins - 1)
                plsc.addupdate_scatter(hist_v, [iota * nbins + digit], ones,
                                       mask=valid)
                return 0

            lax.fori_loop(0, nit, lv_hist, 0)
            d, k_cur = histogram_scan(nbins, k_cur)
            prefix = (prefix << w) | d

            if lvl < len(_LEVELS) - 1:
                def lv_collect(i, off, src=src, dst=dst, sh=sh, m=m,
                               prefix=prefix):
                    b = src[pl.ds(i * _L, _L)]
                    valid = (i * _L + iota) < m
                    match = valid & (
                        lax.shift_right_logical(b, jnp.int32(sh)) == prefix)
                    mi = match.astype(jnp.int32)
                    cs = plsc.cumsum(mi)
                    pos = off + cs - mi
                    plsc.store_scatter(dst, [pos], b, mask=match)
                    return off + jnp.sum(mi)

                m = lax.fori_loop(0, nit, lv_collect, jnp.int32(0))
                src, dst = dst, src

        thresh = prefix

        def apply_chunk(i, _):
            x = row_v[pl.ds(i * _L, _L)]
            bits = plsc.bitcast(x, jnp.int32) & jnp.int32(0x7FFFFFFF)
            keep = bits >= thresh
            row_v[pl.ds(i * _L, _L)] = jnp.where(keep, x, jnp.float32(0.0))
            return 0

        lax.fori_loop(0, _NCH, apply_chunk, 0)
        pltpu.sync_copy(row_v, out_hbm.at[row])
        return 0

    lax.fori_loop(0, _RPW, do_row, 0)


@functools.partial(jax.jit, static_argnames=())
def _run(scores, kk):
    mesh = plsc.VectorSubcoreMesh(core_axis_name="c", subcore_axis_name="s")
    fn = functools.partial(
        pl.kernel,
        mesh=mesh,
        out_type=jax.ShapeDtypeStruct((_B, _N), jnp.float32),
        compiler_params=pltpu.CompilerParams(needs_layout_passes=False),
        scratch_types=[
            pltpu.VMEM((_N,), jnp.float32),
            pltpu.VMEM((_N + _L,), jnp.int32),
            pltpu.VMEM((_N + _L,), jnp.int32),
            pltpu.VMEM((_MAXBINS * _L,), jnp.int32),
            pltpu.VMEM((_MAXBINS,), jnp.int32),
            pltpu.VMEM((_L,), jnp.int32),
        ],
    )(_body)
    return fn(scores, kk)


def kernel(scores, k):
    kk = jnp.full((_L,), k, dtype=jnp.int32)
    return _run(scores, kk)


# speedup vs baseline: 2.1715x; 2.1715x over previous
import functools

import jax
import jax.numpy as jnp
from jax import lax
from jax.experimental import pallas as pl
from jax.experimental.pallas import tpu as pltpu
from jax.experimental.pallas import tpu_sc as plsc

_B = 128
_N = 32768
_NW = 32
_RPW = _B // _NW
_L = 16
_NCH = _N // _L

_LEVELS = ((24, 7), (18, 6), (12, 6), (6, 6), (0, 6))
_MAXBINS = 1 << 7


def _body(scores_hbm, k_hbm, out_hbm, row_v, coll_v, hist_v, tot_v, kv_v):
    nc = 2
    wid = lax.axis_index("s") * nc + lax.axis_index("c")
    pltpu.sync_copy(k_hbm, kv_v)
    k0 = kv_v[pl.ds(0, _L)][0]
    iota = lax.iota(jnp.int32, _L)
    ones = jnp.ones((_L,), jnp.int32)
    msk31 = jnp.int32(0x7FFFFFFF)

    def histogram_scan(nbins, k_cur):
        nch = nbins // _L

        def merge_chunk(c, _):
            acc = hist_v[pl.ds(c * _L, _L)]
            for l in range(1, _L):
                acc = acc + hist_v[pl.ds(l * nbins + c * _L, _L)]
            tot_v[pl.ds(c * _L, _L)] = acc
            return 0

        lax.fori_loop(0, nch, merge_chunk, 0, unroll=True)

        def scan_chunk(j, carry):
            after, best_d, best_k = carry
            c = nch - 1 - j
            v = tot_v[pl.ds(c * _L, _L)]
            cs = plsc.cumsum(v)
            tot = cs[_L - 1]
            suffix_incl = after + tot - cs + v
            feas = suffix_incl >= k_cur
            lane = jnp.sum(feas.astype(jnp.int32)) - 1
            above = jnp.sum(jnp.where(iota == lane, suffix_incl - v,
                                      jnp.int32(0)))
            found = (lane >= 0) & (best_d < 0)
            best_d = jnp.where(found, c * _L + lane, best_d)
            best_k = jnp.where(found, k_cur - above, best_k)
            return after + tot, best_d, best_k

        _, d, k_new = lax.fori_loop(
            0, nch, scan_chunk,
            (jnp.int32(0), jnp.int32(-1), jnp.int32(0)))
        return d, k_new

    def zero_hist(nbins):
        @plsc.parallel_loop(0, nbins, unroll=8)
        def _(i):
            hist_v[pl.ds(i * _L, _L)] = jnp.zeros((_L,), jnp.int32)

    def do_row(r, _):
        row = wid * _RPW + r
        pltpu.sync_copy(scores_hbm.at[row], row_v)

        sh0, w0 = _LEVELS[0]
        nbins0 = 1 << w0
        zero_hist(nbins0)
        lane_base = iota * nbins0

        @plsc.parallel_loop(0, _NCH, unroll=8)
        def _(i):
            x = row_v[pl.ds(i * _L, _L)]
            bits = plsc.bitcast(x, jnp.int32) & msk31
            digit = lax.shift_right_logical(bits, jnp.int32(sh0))
            plsc.addupdate_scatter(hist_v, [lane_base + digit], ones)

        d1, k_cur = histogram_scan(nbins0, k0)
        prefix = d1

        @plsc.parallel_loop(0, _NCH, unroll=8,
                            carry=jnp.zeros((_L,), jnp.int32))
        def off_final(i, off):
            x = row_v[pl.ds(i * _L, _L)]
            bits = plsc.bitcast(x, jnp.int32) & msk31
            match = lax.shift_right_logical(bits, jnp.int32(sh0)) == prefix
            mi = match.astype(jnp.int32)
            cs = plsc.cumsum(mi)
            pos = off + cs - mi
            plsc.store_scatter(coll_v, [pos], bits, mask=match)
            return off + plsc.all_reduce_population_count(match)

        m = off_final[0]

        for lvl in range(1, len(_LEVELS)):
            sh, w = _LEVELS[lvl]
            nbins = 1 << w
            zero_hist(nbins)
            nit = (m + (_L - 1)) // _L
            lane_base2 = iota * nbins

            def lv_hist(i, _, sh=sh, nbins=nbins, m=m, lane_base2=lane_base2):
                b = coll_v[pl.ds(i * _L, _L)]
                valid = (i * _L + iota) < m
                digit = lax.shift_right_logical(b, jnp.int32(sh)) \
                    & jnp.int32(nbins - 1)
                plsc.addupdate_scatter(hist_v, [lane_base2 + digit], ones,
                                       mask=valid)
                return 0

            lax.fori_loop(0, nit, lv_hist, 0)
            d, k_cur = histogram_scan(nbins, k_cur)
            prefix = (prefix << w) | d

            if lvl < len(_LEVELS) - 1:
                def lv_collect(i, off, sh=sh, m=m, prefix=prefix):
                    b = coll_v[pl.ds(i * _L, _L)]
                    valid = (i * _L + iota) < m
                    match = valid & (
                        lax.shift_right_logical(b, jnp.int32(sh)) == prefix)
                    mi = match.astype(jnp.int32)
                    cs = plsc.cumsum(mi)
                    pos = off + cs - mi
                    plsc.store_scatter(coll_v, [pos], b, mask=match)
                    return off + cs[_L - 1]

                m = lax.fori_loop(0, nit, lv_collect, jnp.int32(0))

        thresh = prefix

        @plsc.parallel_loop(0, _NCH, unroll=8)
        def _(i):
            x = row_v[pl.ds(i * _L, _L)]
            bits = plsc.bitcast(x, jnp.int32) & msk31
            keep = bits >= thresh
            row_v[pl.ds(i * _L, _L)] = jnp.where(keep, x, jnp.float32(0.0))

        pltpu.sync_copy(row_v, out_hbm.at[row])
        return 0

    lax.fori_loop(0, _RPW, do_row, 0)


@jax.jit
def _run(scores, kk):
    mesh = plsc.VectorSubcoreMesh(core_axis_name="c", subcore_axis_name="s")
    fn = functools.partial(
        pl.kernel,
        mesh=mesh,
        out_type=jax.ShapeDtypeStruct((_B, _N), jnp.float32),
        compiler_params=pltpu.CompilerParams(needs_layout_passes=False),
        scratch_types=[
            pltpu.VMEM((_N,), jnp.float32),
            pltpu.VMEM((_N + _L,), jnp.int32),
            pltpu.VMEM((_MAXBINS * _L,), jnp.int32),
            pltpu.VMEM((_MAXBINS,), jnp.int32),
            pltpu.VMEM((_L,), jnp.int32),
        ],
    )(_body)
    return fn(scores, kk)


def kernel(scores, k):
    kk = jnp.full((_L,), k, dtype=jnp.int32)
    return _run(scores, kk)


# speedup vs baseline: 2.3179x; 1.0674x over previous
import functools

import jax
import jax.numpy as jnp
from jax import lax
from jax.experimental import pallas as pl
from jax.experimental.pallas import tpu as pltpu
from jax.experimental.pallas import tpu_sc as plsc

_B = 128
_N = 32768
_NW = 32
_RPW = _B // _NW
_L = 16
_NCH = _N // _L

_LEVELS = ((24, 7), (18, 6), (12, 6), (6, 6), (0, 6))
_MAXBINS = 1 << 7


def _body(scores_hbm, k_hbm, out_hbm, row_v, coll_v, hist_v, tot_v, kv_v):
    nc = 2
    wid = lax.axis_index("s") * nc + lax.axis_index("c")
    pltpu.sync_copy(k_hbm, kv_v)
    k0 = kv_v[pl.ds(0, _L)][0]
    iota = lax.iota(jnp.int32, _L)
    ones = jnp.ones((_L,), jnp.int32)
    msk31 = jnp.int32(0x7FFFFFFF)

    def histogram_scan(nbins, k_cur):
        nch = nbins // _L

        def merge_chunk(c, _):
            acc = hist_v[pl.ds(c * _L, _L)]
            for l in range(1, _L):
                acc = acc + hist_v[pl.ds(l * nbins + c * _L, _L)]
            tot_v[pl.ds(c * _L, _L)] = acc
            return 0

        lax.fori_loop(0, nch, merge_chunk, 0, unroll=True)

        def scan_chunk(j, carry):
            after, best_d, best_k = carry
            c = nch - 1 - j
            v = tot_v[pl.ds(c * _L, _L)]
            cs = plsc.cumsum(v)
            tot = cs[_L - 1]
            suffix_incl = after + tot - cs + v
            feas = suffix_incl >= k_cur
            lane = jnp.sum(feas.astype(jnp.int32)) - 1
            above = jnp.sum(jnp.where(iota == lane, suffix_incl - v,
                                      jnp.int32(0)))
            found = (lane >= 0) & (best_d < 0)
            best_d = jnp.where(found, c * _L + lane, best_d)
            best_k = jnp.where(found, k_cur - above, best_k)
            return after + tot, best_d, best_k

        _, d, k_new = lax.fori_loop(
            0, nch, scan_chunk,
            (jnp.int32(0), jnp.int32(-1), jnp.int32(0)))
        return d, k_new

    def zero_hist(nbins):
        @plsc.parallel_loop(0, nbins, unroll=8)
        def _(i):
            hist_v[pl.ds(i * _L, _L)] = jnp.zeros((_L,), jnp.int32)

    def do_row(r, _):
        row = wid * _RPW + r
        pltpu.sync_copy(scores_hbm.at[row], row_v)

        sh0, w0 = _LEVELS[0]
        nbins0 = 1 << w0
        zero_hist(nbins0)
        lane_base = iota * nbins0

        @plsc.parallel_loop(0, _NCH, unroll=16)
        def _(i):
            x = row_v[pl.ds(i * _L, _L)]
            bits = plsc.bitcast(x, jnp.int32) & msk31
            digit = lax.shift_right_logical(bits, jnp.int32(sh0))
            plsc.addupdate_scatter(hist_v, [lane_base + digit], ones)

        d1, k_cur = histogram_scan(nbins0, k0)
        prefix = d1

        @plsc.parallel_loop(0, _NCH, unroll=8, carry=jnp.int32(0))
        def off_final(i, off):
            x = row_v[pl.ds(i * _L, _L)]
            bits = plsc.bitcast(x, jnp.int32) & msk31
            match = lax.shift_right_logical(bits, jnp.int32(sh0)) == prefix
            plsc.store_compressed(coll_v.at[pl.ds(off, _L)], bits,
                                  mask=match)
            return off + plsc.all_reduce_population_count(match)[0]

        m = off_final

        for lvl in range(1, len(_LEVELS)):
            sh, w = _LEVELS[lvl]
            nbins = 1 << w
            zero_hist(nbins)
            nit = (m + (_L - 1)) // _L
            lane_base2 = iota * nbins

            def lv_hist(i, _, sh=sh, nbins=nbins, m=m, lane_base2=lane_base2):
                b = coll_v[pl.ds(i * _L, _L)]
                valid = (i * _L + iota) < m
                digit = lax.shift_right_logical(b, jnp.int32(sh)) \
                    & jnp.int32(nbins - 1)
                plsc.addupdate_scatter(hist_v, [lane_base2 + digit], ones,
                                       mask=valid)
                return 0

            lax.fori_loop(0, nit, lv_hist, 0)
            d, k_cur = histogram_scan(nbins, k_cur)
            prefix = (prefix << w) | d

            if lvl < len(_LEVELS) - 1:
                def lv_collect(i, off, sh=sh, m=m, prefix=prefix):
                    b = coll_v[pl.ds(i * _L, _L)]
                    valid = (i * _L + iota) < m
                    match = valid & (
                        lax.shift_right_logical(b, jnp.int32(sh)) == prefix)
                    plsc.store_compressed(coll_v.at[pl.ds(off, _L)], b,
                                          mask=match)
                    return off + plsc.all_reduce_population_count(match)[0]

                m = lax.fori_loop(0, nit, lv_collect, jnp.int32(0))

        thresh = prefix

        @plsc.parallel_loop(0, _NCH, unroll=16)
        def _(i):
            x = row_v[pl.ds(i * _L, _L)]
            bits = plsc.bitcast(x, jnp.int32) & msk31
            keep = bits >= thresh
            row_v[pl.ds(i * _L, _L)] = jnp.where(keep, x, jnp.float32(0.0))

        pltpu.sync_copy(row_v, out_hbm.at[row])
        return 0

    lax.fori_loop(0, _RPW, do_row, 0)


@jax.jit
def _run(scores, kk):
    mesh = plsc.VectorSubcoreMesh(core_axis_name="c", subcore_axis_name="s")
    fn = functools.partial(
        pl.kernel,
        mesh=mesh,
        out_type=jax.ShapeDtypeStruct((_B, _N), jnp.float32),
        compiler_params=pltpu.CompilerParams(needs_layout_passes=False),
        scratch_types=[
            pltpu.VMEM((_N,), jnp.float32),
            pltpu.VMEM((_N + _L,), jnp.int32),
            pltpu.VMEM((_MAXBINS * _L,), jnp.int32),
            pltpu.VMEM((_MAXBINS,), jnp.int32),
            pltpu.VMEM((_L,), jnp.int32),
        ],
    )(_body)
    return fn(scores, kk)


def kernel(scores, k):
    kk = jnp.full((_L,), k, dtype=jnp.int32)
    return _run(scores, kk)


# speedup vs baseline: 2.3263x; 1.0036x over previous
import functools

import jax
import jax.numpy as jnp
from jax import lax
from jax.experimental import pallas as pl
from jax.experimental.pallas import tpu as pltpu
from jax.experimental.pallas import tpu_sc as plsc

_B = 128
_N = 32768
_NW = 32
_RPW = _B // _NW
_L = 16
_NCH = _N // _L

_LEVELS = ((24, 7), (18, 6), (12, 6), (6, 6), (0, 6))
_MAXBINS = 1 << 7


def _body(scores_hbm, k_hbm, out_hbm, rowa_v, rowb_v, coll_v, hist_v, tot_v,
          kv_v, sia, sib, soa, sob):
    nc = 2
    wid = lax.axis_index("s") * nc + lax.axis_index("c")
    pltpu.sync_copy(k_hbm, kv_v)
    k0 = kv_v[pl.ds(0, _L)][0]
    iota = lax.iota(jnp.int32, _L)
    ones = jnp.ones((_L,), jnp.int32)
    msk31 = jnp.int32(0x7FFFFFFF)

    def histogram_scan(nbins, k_cur):
        nch = nbins // _L

        def merge_chunk(c, _):
            acc = hist_v[pl.ds(c * _L, _L)]
            for l in range(1, _L):
                acc = acc + hist_v[pl.ds(l * nbins + c * _L, _L)]
            tot_v[pl.ds(c * _L, _L)] = acc
            return 0

        lax.fori_loop(0, nch, merge_chunk, 0, unroll=True)

        def scan_chunk(j, carry):
            after, best_d, best_k = carry
            c = nch - 1 - j
            v = tot_v[pl.ds(c * _L, _L)]
            cs = plsc.cumsum(v)
            tot = cs[_L - 1]
            suffix_incl = after + tot - cs + v
            feas = suffix_incl >= k_cur
            lane = jnp.sum(feas.astype(jnp.int32)) - 1
            above = jnp.sum(jnp.where(iota == lane, suffix_incl - v,
                                      jnp.int32(0)))
            found = (lane >= 0) & (best_d < 0)
            best_d = jnp.where(found, c * _L + lane, best_d)
            best_k = jnp.where(found, k_cur - above, best_k)
            return after + tot, best_d, best_k

        _, d, k_new = lax.fori_loop(
            0, nch, scan_chunk,
            (jnp.int32(0), jnp.int32(-1), jnp.int32(0)))
        return d, k_new

    def zero_hist(nbins):
        @plsc.parallel_loop(0, nbins, unroll=8)
        def _(i):
            hist_v[pl.ds(i * _L, _L)] = jnp.zeros((_L,), jnp.int32)

    def do_row(row, row_v):
        sh0, w0 = _LEVELS[0]
        nbins0 = 1 << w0
        zero_hist(nbins0)
        lane_base = iota * nbins0

        @plsc.parallel_loop(0, _NCH, unroll=16)
        def _(i):
            x = row_v[pl.ds(i * _L, _L)]
            bits = plsc.bitcast(x, jnp.int32) & msk31
            digit = lax.shift_right_logical(bits, jnp.int32(sh0))
            plsc.addupdate_scatter(hist_v, [lane_base + digit], ones)

        d1, k_cur = histogram_scan(nbins0, k0)
        prefix = d1

        @plsc.parallel_loop(0, _NCH, unroll=8, carry=jnp.int32(0))
        def off_final(i, off):
            x = row_v[pl.ds(i * _L, _L)]
            bits = plsc.bitcast(x, jnp.int32) & msk31
            match = lax.shift_right_logical(bits, jnp.int32(sh0)) == prefix
            plsc.store_compressed(coll_v.at[pl.ds(off, _L)], bits,
                                  mask=match)
            return off + plsc.all_reduce_population_count(match)[0]

        m = off_final

        for lvl in range(1, len(_LEVELS)):
            sh, w = _LEVELS[lvl]
            nbins = 1 << w
            zero_hist(nbins)
            nit = (m + (_L - 1)) // _L
            lane_base2 = iota * nbins

            def lv_hist(i, _, sh=sh, nbins=nbins, m=m, lane_base2=lane_base2):
                b = coll_v[pl.ds(i * _L, _L)]
                valid = (i * _L + iota) < m
                digit = lax.shift_right_logical(b, jnp.int32(sh)) \
                    & jnp.int32(nbins - 1)
                plsc.addupdate_scatter(hist_v, [lane_base2 + digit], ones,
                                       mask=valid)
                return 0

            lax.fori_loop(0, nit, lv_hist, 0)
            d, k_cur = histogram_scan(nbins, k_cur)
            prefix = (prefix << w) | d

            if lvl < len(_LEVELS) - 1:
                def lv_collect(i, off, sh=sh, m=m, prefix=prefix):
                    b = coll_v[pl.ds(i * _L, _L)]
                    valid = (i * _L + iota) < m
                    match = valid & (
                        lax.shift_right_logical(b, jnp.int32(sh)) == prefix)
                    plsc.store_compressed(coll_v.at[pl.ds(off, _L)], b,
                                          mask=match)
                    return off + plsc.all_reduce_population_count(match)[0]

                m = lax.fori_loop(0, nit, lv_collect, jnp.int32(0))

        thresh = prefix

        @plsc.parallel_loop(0, _NCH, unroll=16)
        def _(i):
            x = row_v[pl.ds(i * _L, _L)]
            bits = plsc.bitcast(x, jnp.int32) & msk31
            keep = bits >= thresh
            row_v[pl.ds(i * _L, _L)] = jnp.where(keep, x, jnp.float32(0.0))

    base = wid * _RPW
    bufs = (rowa_v, rowb_v)
    in_sems = (sia, sib)
    out_sems = (soa, sob)
    copies_in = {}
    copies_out = {}
    copies_in[0] = pltpu.async_copy(scores_hbm.at[base], rowa_v, sia)
    for r in range(_RPW):
        buf = bufs[r % 2]
        copies_in[r].wait()
        if r >= 2:
            copies_out[r - 2].wait()
        if r + 1 < _RPW:
            nbuf = bufs[(r + 1) % 2]
            copies_in[r + 1] = pltpu.async_copy(
                scores_hbm.at[base + (r + 1)], nbuf, in_sems[(r + 1) % 2])
        do_row(base + r, buf)
        copies_out[r] = pltpu.async_copy(buf, out_hbm.at[base + r],
                                         out_sems[r % 2])
    copies_out[_RPW - 2].wait()
    copies_out[_RPW - 1].wait()


@jax.jit
def _run(scores, kk):
    mesh = plsc.VectorSubcoreMesh(core_axis_name="c", subcore_axis_name="s")
    fn = functools.partial(
        pl.kernel,
        mesh=mesh,
        out_type=jax.ShapeDtypeStruct((_B, _N), jnp.float32),
        compiler_params=pltpu.CompilerParams(needs_layout_passes=False),
        scratch_types=[
            pltpu.VMEM((_N,), jnp.float32),
            pltpu.VMEM((_N,), jnp.float32),
            pltpu.VMEM((_N + _L,), jnp.int32),
            pltpu.VMEM((_MAXBINS * _L,), jnp.int32),
            pltpu.VMEM((_MAXBINS,), jnp.int32),
            pltpu.VMEM((_L,), jnp.int32),
            pltpu.SemaphoreType.DMA,
            pltpu.SemaphoreType.DMA,
            pltpu.SemaphoreType.DMA,
            pltpu.SemaphoreType.DMA,
        ],
    )(_body)
    return fn(scores, kk)


def kernel(scores, k):
    kk = jnp.full((_L,), k, dtype=jnp.int32)
    return _run(scores, kk)
